# Initial kernel scaffold; baseline (speedup 1.0000x reference)
import jax
import jax.numpy as jnp
from jax.experimental import pallas as pl


def kernel(location_indices, time_slices, grid_assign, param_0, param_1, param_2, param_3, param_4, param_5, param_6, param_7):
    raise NotImplementedError("write your pallas kernel here")


# speedup vs baseline: 3.3547x; 3.3547x over previous
import functools

import jax
import jax.numpy as jnp
from jax import lax
from jax.experimental import pallas as pl
from jax.experimental.pallas import tpu as pltpu
from jax.experimental.pallas import tpu_sc as plsc

_HEIGHT = 8
_TOPICS = 64
_NTIME = 24
_BATCH = 16384
_NC = 2
_NS = 16
_NW = _NC * _NS
_BPW = _BATCH // _NW
_CHUNK = 128
_NCHUNK = _BPW // _CHUNK
_LANES = 16


def _body(loc_hbm, t_hbm, g7_hbm,
          p0, p1, p2, p3, p4, p5, p6, p7,
          out_hbm,
          loc_v, t_v, c7_v, ridx_v, bufs_v, sem):
    params = (p0, p1, p2, p3, p4, p5, p6, p7)
    wid = lax.axis_index("s") * _NC + lax.axis_index("c")
    rb = wid * _NCHUNK

    pltpu.sync_copy(loc_hbm.at[pl.ds(rb, _NCHUNK)], loc_v)
    pltpu.sync_copy(t_hbm.at[pl.ds(rb, _NCHUNK)], t_v)

    cps = [
        pltpu.async_copy(g7_hbm.at[loc_v.at[j]], c7_v.at[j], sem)
        for j in range(_NCHUNK)
    ]
    for cp in cps:
        cp.wait()

    for j in range(_NCHUNK):
        def ridx_body(v, _, j=j):
            s = pl.ds(v * _LANES, _LANES)
            c7 = c7_v[j, s]
            t = t_v[j, s]
            li = lax.shift_right_logical(c7, 7)
            lo = lax.bitwise_and(c7, 127)
            ridx_v[0, j, s] = t
            for h in range(1, _HEIGHT):
                sh = 7 - h
                cell = lax.bitwise_or(
                    lax.shift_left(lax.shift_right_logical(li, sh), h),
                    lax.shift_right_logical(lo, sh))
                ridx_v[h, j, s] = cell * _NTIME + t
            return 0
        lax.fori_loop(0, _CHUNK // _LANES, ridx_body, 0)

    for j in range(_NCHUNK):
        cps = [
            pltpu.async_copy(params[h].at[ridx_v.at[h, j]], bufs_v.at[h], sem)
            for h in range(_HEIGHT)
        ]
        for cp in cps:
            cp.wait()

        def acc_body(r, _):
            for c in range(_TOPICS // _LANES):
                s = pl.ds(c * _LANES, _LANES)
                x = bufs_v[0, r, s]
                for h in range(1, _HEIGHT):
                    x = x + bufs_v[h, r, s]
                bufs_v[0, r, s] = x
            return 0
        lax.fori_loop(0, _CHUNK, acc_body, 0)

        pltpu.sync_copy(bufs_v.at[0],
                        out_hbm.at[pl.ds(wid * _BPW + j * _CHUNK, _CHUNK)])


def kernel(location_indices, time_slices, grid_assign,
           param_0, param_1, param_2, param_3,
           param_4, param_5, param_6, param_7):
    loc2 = location_indices.astype(jnp.int32).reshape(_BATCH // _CHUNK, _CHUNK)
    t2 = time_slices.astype(jnp.int32).reshape(_BATCH // _CHUNK, _CHUNK)
    g7 = grid_assign[_HEIGHT - 1].astype(jnp.int32)
    flat = [p.reshape(-1, _TOPICS) for p in
            (param_0, param_1, param_2, param_3,
             param_4, param_5, param_6, param_7)]

    mesh = plsc.VectorSubcoreMesh(core_axis_name="c", subcore_axis_name="s")
    run = functools.partial(
        pl.kernel,
        mesh=mesh,
        compiler_params=pltpu.CompilerParams(use_tc_tiling_on_sc=False),
        out_type=jax.ShapeDtypeStruct((_BATCH, _TOPICS), jnp.float32),
        scratch_types=[
            pltpu.VMEM((_NCHUNK, _CHUNK), jnp.int32),
            pltpu.VMEM((_NCHUNK, _CHUNK), jnp.int32),
            pltpu.VMEM((_NCHUNK, _CHUNK), jnp.int32),
            pltpu.VMEM((_HEIGHT, _NCHUNK, _CHUNK), jnp.int32),
            pltpu.VMEM((_HEIGHT, _CHUNK, _TOPICS), jnp.float32),
            pltpu.SemaphoreType.DMA,
        ],
    )(_body)
    return run(loc2, t2, g7, *flat)
